# baseline (speedup 1.0000x reference)
import functools

import jax
import jax.numpy as jnp
from jax import lax
from jax.experimental import pallas as pl
from jax.experimental.pallas import tpu as pltpu
from jax.experimental.pallas import tpu_sc as plsc

D = 128
NC, NS = 2, 16
CH = 80
L = 16


def _sc_scatter(x, src3, dst3, zrows):
    n = x.shape[0]
    rps = (n // NS) // 8 * 8
    tail = n - (NS - 1) * rps
    chunks = src3.shape[1]
    mesh = plsc.VectorSubcoreMesh(core_axis_name="c", subcore_axis_name="s",
                                  num_cores=NC, num_subcores=NS)

    @functools.partial(
        pl.kernel,
        out_type=(jax.ShapeDtypeStruct((NC * n, D), jnp.float32),
                  jax.ShapeDtypeStruct((n // 1000, NC * NS, 1000),
                                       jnp.float32)),
        mesh=mesh,
        scratch_types=[
            pltpu.VMEM((chunks, CH), jnp.int32),
            pltpu.VMEM((chunks, CH), jnp.int32),
            pltpu.VMEM((CH, D), jnp.float32),
            pltpu.VMEM((CH, D), jnp.float32),
            pltpu.VMEM((n,), jnp.float32),
            pltpu.VMEM_SHARED((n, D), jnp.float32),
            pltpu.SemaphoreType.DMA,
            pltpu.SemaphoreType.DMA,
            pltpu.SemaphoreType.DMA,
            pltpu.SemaphoreType.DMA,
        ],
        compiler_params=pltpu.CompilerParams(use_tc_tiling_on_sc=False,
                                             needs_layout_passes=False),
    )
    def k(x_hbm, src_hbm, dst_hbm, z_hbm, out_hbm, cnt_hbm, src_v, dst_v,
          rows0, rows1, cnt_v, s_sh, sem0, sem1, sems0, sems1):
        c = lax.axis_index("c")
        s = lax.axis_index("s")
        wid = c * NS + s
        pltpu.sync_copy(src_hbm.at[wid], src_v)
        pltpu.sync_copy(dst_hbm.at[wid], dst_v)
        @pl.when((s < NS - 1) & (c == 0))
        def _x0():
            pltpu.sync_copy(x_hbm.at[pl.ds(s * rps, rps)],
                            s_sh.at[pl.ds(s * rps, rps)])

        @pl.when((s == NS - 1) & (c == 0))
        def _x1():
            pltpu.sync_copy(x_hbm.at[pl.ds((NS - 1) * rps, tail)],
                            s_sh.at[pl.ds((NS - 1) * rps, tail)])

        @pl.when((s < NS - 1) & (c == 1))
        def _z0():
            pltpu.sync_copy(z_hbm.at[pl.ds(0, rps)],
                            s_sh.at[pl.ds(s * rps, rps)])

        @pl.when((s == NS - 1) & (c == 1))
        def _z1():
            pltpu.sync_copy(z_hbm, s_sh.at[pl.ds((NS - 1) * rps, tail)])

        fill16 = jnp.where(wid == 0, jnp.full((L,), 1.0, jnp.float32),
                           jnp.zeros((L,), jnp.float32))

        def zc(i, carry):
            cnt_v[pl.ds(i * L, L)] = fill16
            return carry

        lax.fori_loop(0, n // L, zc, 0)
        plsc.subcore_barrier()

        one16 = jnp.full((L,), 1.0, jnp.float32)
        rem = CH % L
        tail_mask = lax.iota(jnp.int32, L) >= (L - rem)

        def count(i):
            for kk in range(CH // L):
                idx = dst_v[i, pl.ds(kk * L, L)]
                plsc.addupdate_scatter(cnt_v, [idx], one16)
            if rem:
                idx = dst_v[i, pl.ds(CH - L, L)]
                plsc.addupdate_scatter(cnt_v, [idx], one16, mask=tail_mask)

        pltpu.async_copy(x_hbm.at[src_v.at[0]], rows0, sem0)
        pltpu.async_copy(x_hbm.at[src_v.at[1]], rows1, sem1)

        def scat(i, rows, sems):
            count(i)
            pltpu.async_copy(rows, s_sh.at[dst_v.at[i]], sems, add=True)

        def wait_g(i, rows, sem):
            pltpu.make_async_copy(x_hbm.at[src_v.at[i]], rows, sem).wait()

        def wait_s(i, rows, sems):
            pltpu.make_async_copy(rows, s_sh.at[dst_v.at[i]], sems).wait()

        def pair(j, carry):
            i = 2 * j
            wait_g(i, rows0, sem0)
            scat(i, rows0, sems0)
            wait_g(i + 1, rows1, sem1)
            scat(i + 1, rows1, sems1)
            wait_s(i, rows0, sems0)
            pltpu.async_copy(x_hbm.at[src_v.at[i + 2]], rows0, sem0)
            wait_s(i + 1, rows1, sems1)
            pltpu.async_copy(x_hbm.at[src_v.at[i + 3]], rows1, sem1)
            return carry

        lax.fori_loop(0, (chunks - 3) // 2, pair, 0)
        i0 = chunks - 3
        wait_g(i0, rows0, sem0)
        scat(i0, rows0, sems0)
        wait_g(i0 + 1, rows1, sem1)
        scat(i0 + 1, rows1, sems1)
        wait_s(i0, rows0, sems0)
        pltpu.async_copy(x_hbm.at[src_v.at[chunks - 1]], rows0, sem0)
        wait_g(chunks - 1, rows0, sem0)
        scat(chunks - 1, rows0, sems0)
        wait_s(i0 + 1, rows1, sems1)
        wait_s(chunks - 1, rows0, sems0)
        plsc.subcore_barrier()

        for b in range(n // 1000):
            pltpu.sync_copy(cnt_v.at[pl.ds(b * 1000, 1000)],
                            cnt_hbm.at[b, wid])

        @pl.when(s < NS - 1)
        def _r0():
            pltpu.sync_copy(s_sh.at[pl.ds(s * rps, rps)],
                            out_hbm.at[pl.ds(c * n + s * rps, rps)])

        @pl.when(s == NS - 1)
        def _r1():
            pltpu.sync_copy(s_sh.at[pl.ds((NS - 1) * rps, tail)],
                            out_hbm.at[pl.ds(c * n + (NS - 1) * rps, tail)])

    return k(x, src3, dst3, zrows)


def _tc_dense(sacc, cnt, wlt, bl, w1t, b1r, w2, b2r):
    n = sacc.shape[0] // NC
    nw = cnt.shape[1]
    cb = cnt.shape[2]
    br = 2000
    g = n // br
    cpb = br // cb

    def body(s0, s1, cnt_r, wlt_r, bl_r, w1t_r, b1_r, w2_r, b2_r, out, acc):
        i = pl.program_id(0)

        @pl.when(i == 0)
        def _init():
            acc[...] = jnp.zeros_like(acc)

        ones_col = jnp.ones((nw, 1), jnp.float32)
        cnt_col = jnp.concatenate(
            [lax.dot_general(cnt_r[b], ones_col, (((0,), (0,)), ((), ())),
                             preferred_element_type=jnp.float32)
             for b in range(cpb)], axis=0)
        feat = s0[...] + s1[...]
        agg = feat / cnt_col
        t = jnp.dot(agg, wlt_r[...], preferred_element_type=jnp.float32) + bl_r[...]
        h = jnp.maximum(
            jnp.dot(t, w1t_r[...], preferred_element_type=jnp.float32) + b1_r[...],
            0.0)
        acc[...] = acc[...] + jnp.sum(h, axis=0, keepdims=True)

        @pl.when(i == g - 1)
        def _fin():
            out[...] = (lax.dot_general(acc[...], w2_r[...],
                                        (((1,), (1,)), ((), ())),
                                        preferred_element_type=jnp.float32)
                        + b2_r[...] * float(n))

    full = lambda i: (0, 0)
    ow = w2.shape[0]
    return pl.pallas_call(
        body,
        grid=(g,),
        in_specs=[
            pl.BlockSpec((br, D), lambda i: (i, 0)),
            pl.BlockSpec((br, D), lambda i, _g=g: (i + _g, 0)),
            pl.BlockSpec((cpb, nw, cb), lambda i: (i, 0, 0)),
            pl.BlockSpec((D, D), full),
            pl.BlockSpec((1, D), full),
            pl.BlockSpec((D, D), full),
            pl.BlockSpec((1, D), full),
            pl.BlockSpec((ow, D), full),
            pl.BlockSpec((1, ow), full),
        ],
        out_specs=pl.BlockSpec((1, ow), full),
        out_shape=jax.ShapeDtypeStruct((1, ow), jnp.float32),
        scratch_shapes=[pltpu.VMEM((1, D), jnp.float32)],
    )(sacc, sacc, cnt, wlt, bl, w1t, b1r, w2, b2r)


def kernel(x, edge_index, W_lin, b_lin, W1, b1, W2, b2):
    n, d = x.shape
    e = edge_index.shape[1]
    out_w = W2.shape[0]
    nw = NC * NS
    src3 = edge_index[0].reshape(nw, e // (nw * CH), CH)
    dst3 = edge_index[1].reshape(nw, e // (nw * CH), CH)
    zrows = jnp.zeros((n - (NS - 1) * ((n // NS) // 8 * 8), D), jnp.float32)
    sacc, cnt = _sc_scatter(x, src3, dst3, zrows)
    out_row = _tc_dense(sacc, cnt, W_lin.T, b_lin.reshape(1, D), W1.T,
                        b1.reshape(1, D), W2, b2.reshape(1, out_w))
    return out_row[0]


# speedup vs baseline: 1.2242x; 1.2242x over previous
import functools

import jax
import jax.numpy as jnp
from jax import lax
from jax.experimental import pallas as pl
from jax.experimental.pallas import tpu as pltpu
from jax.experimental.pallas import tpu_sc as plsc

D = 128
NC, NS = 2, 16
CH = 80
L = 16


def _sc_scatter(x, src3, dst3, zrows):
    n = x.shape[0]
    rps = (n // NS) // 8 * 8
    tail = n - (NS - 1) * rps
    chunks = src3.shape[1]
    mesh = plsc.VectorSubcoreMesh(core_axis_name="c", subcore_axis_name="s",
                                  num_cores=NC, num_subcores=NS)

    @functools.partial(
        pl.kernel,
        out_type=(jax.ShapeDtypeStruct((NC * n, D), jnp.float32),
                  jax.ShapeDtypeStruct((n // 1000, NC * NS, 1000),
                                       jnp.float32)),
        mesh=mesh,
        scratch_types=[
            pltpu.VMEM((chunks, CH), jnp.int32),
            pltpu.VMEM((chunks, CH), jnp.int32),
            pltpu.VMEM((CH, D), jnp.float32),
            pltpu.VMEM((CH, D), jnp.float32),
            pltpu.VMEM((n,), jnp.float32),
            pltpu.VMEM_SHARED((n, D), jnp.float32),
            pltpu.SemaphoreType.DMA,
            pltpu.SemaphoreType.DMA,
            pltpu.SemaphoreType.DMA,
            pltpu.SemaphoreType.DMA,
        ],
        compiler_params=pltpu.CompilerParams(use_tc_tiling_on_sc=False,
                                             needs_layout_passes=False),
    )
    def k(x_hbm, src_hbm, dst_hbm, z_hbm, out_hbm, cnt_hbm, src_v, dst_v,
          rows0, rows1, cnt_v, s_sh, sem0, sem1, sems0, sems1):
        c = lax.axis_index("c")
        s = lax.axis_index("s")
        wid = c * NS + s
        pltpu.sync_copy(src_hbm.at[wid], src_v)
        pltpu.sync_copy(dst_hbm.at[wid], dst_v)
        @pl.when((s < NS - 1) & (c == 0))
        def _x0():
            pltpu.sync_copy(x_hbm.at[pl.ds(s * rps, rps)],
                            s_sh.at[pl.ds(s * rps, rps)])

        @pl.when((s == NS - 1) & (c == 0))
        def _x1():
            pltpu.sync_copy(x_hbm.at[pl.ds((NS - 1) * rps, tail)],
                            s_sh.at[pl.ds((NS - 1) * rps, tail)])

        @pl.when((s < NS - 1) & (c == 1))
        def _z0():
            pltpu.sync_copy(z_hbm.at[pl.ds(0, rps)],
                            s_sh.at[pl.ds(s * rps, rps)])

        @pl.when((s == NS - 1) & (c == 1))
        def _z1():
            pltpu.sync_copy(z_hbm, s_sh.at[pl.ds((NS - 1) * rps, tail)])

        fill16 = jnp.where(wid == 0, jnp.full((L,), 1.0, jnp.float32),
                           jnp.zeros((L,), jnp.float32))

        def zc(i, carry):
            cnt_v[pl.ds(i * L, L)] = fill16
            return carry

        lax.fori_loop(0, n // L, zc, 0)
        plsc.subcore_barrier()

        one16 = jnp.full((L,), 1.0, jnp.float32)
        rem = CH % L
        tail_mask = lax.iota(jnp.int32, L) >= (L - rem)

        def count(i):
            for kk in range(CH // L):
                idx = dst_v[i, pl.ds(kk * L, L)]
                plsc.addupdate_scatter(cnt_v, [idx], one16)
            if rem:
                idx = dst_v[i, pl.ds(CH - L, L)]
                plsc.addupdate_scatter(cnt_v, [idx], one16, mask=tail_mask)

        pltpu.async_copy(x_hbm.at[src_v.at[0]], rows0, sem0)
        pltpu.async_copy(x_hbm.at[src_v.at[1]], rows1, sem1)

        def retire(i, rows, sem):
            count(i)
            pltpu.make_async_copy(x_hbm.at[src_v.at[i]], rows, sem).wait()
            pltpu.sync_copy(rows, s_sh.at[dst_v.at[i]], add=True)

        def pair(j, carry):
            i = 2 * j
            retire(i, rows0, sem0)
            pltpu.async_copy(x_hbm.at[src_v.at[i + 2]], rows0, sem0)
            retire(i + 1, rows1, sem1)
            pltpu.async_copy(x_hbm.at[src_v.at[i + 3]], rows1, sem1)
            return carry

        lax.fori_loop(0, (chunks - 3) // 2, pair, 0)
        retire(chunks - 3, rows0, sem0)
        pltpu.async_copy(x_hbm.at[src_v.at[chunks - 1]], rows0, sem0)
        retire(chunks - 2, rows1, sem1)
        retire(chunks - 1, rows0, sem0)
        plsc.subcore_barrier()

        for b in range(n // 1000):
            pltpu.sync_copy(cnt_v.at[pl.ds(b * 1000, 1000)],
                            cnt_hbm.at[b, wid])

        @pl.when(s < NS - 1)
        def _r0():
            pltpu.sync_copy(s_sh.at[pl.ds(s * rps, rps)],
                            out_hbm.at[pl.ds(c * n + s * rps, rps)])

        @pl.when(s == NS - 1)
        def _r1():
            pltpu.sync_copy(s_sh.at[pl.ds((NS - 1) * rps, tail)],
                            out_hbm.at[pl.ds(c * n + (NS - 1) * rps, tail)])

    return k(x, src3, dst3, zrows)


def _tc_dense(sacc, cnt, wlt, bl, w1t, b1r, w2, b2r):
    n = sacc.shape[0] // NC
    nw = cnt.shape[1]
    cb = cnt.shape[2]
    br = 2000
    g = n // br
    cpb = br // cb

    def body(s0, s1, cnt_r, wlt_r, bl_r, w1t_r, b1_r, w2_r, b2_r, out, acc):
        i = pl.program_id(0)

        @pl.when(i == 0)
        def _init():
            acc[...] = jnp.zeros_like(acc)

        ones_col = jnp.ones((nw, 1), jnp.float32)
        cnt_col = jnp.concatenate(
            [lax.dot_general(cnt_r[b], ones_col, (((0,), (0,)), ((), ())),
                             preferred_element_type=jnp.float32)
             for b in range(cpb)], axis=0)
        feat = s0[...] + s1[...]
        agg = feat / cnt_col
        t = jnp.dot(agg, wlt_r[...], preferred_element_type=jnp.float32) + bl_r[...]
        h = jnp.maximum(
            jnp.dot(t, w1t_r[...], preferred_element_type=jnp.float32) + b1_r[...],
            0.0)
        acc[...] = acc[...] + jnp.sum(h, axis=0, keepdims=True)

        @pl.when(i == g - 1)
        def _fin():
            out[...] = (lax.dot_general(acc[...], w2_r[...],
                                        (((1,), (1,)), ((), ())),
                                        preferred_element_type=jnp.float32)
                        + b2_r[...] * float(n))

    full = lambda i: (0, 0)
    ow = w2.shape[0]
    return pl.pallas_call(
        body,
        grid=(g,),
        in_specs=[
            pl.BlockSpec((br, D), lambda i: (i, 0)),
            pl.BlockSpec((br, D), lambda i, _g=g: (i + _g, 0)),
            pl.BlockSpec((cpb, nw, cb), lambda i: (i, 0, 0)),
            pl.BlockSpec((D, D), full),
            pl.BlockSpec((1, D), full),
            pl.BlockSpec((D, D), full),
            pl.BlockSpec((1, D), full),
            pl.BlockSpec((ow, D), full),
            pl.BlockSpec((1, ow), full),
        ],
        out_specs=pl.BlockSpec((1, ow), full),
        out_shape=jax.ShapeDtypeStruct((1, ow), jnp.float32),
        scratch_shapes=[pltpu.VMEM((1, D), jnp.float32)],
    )(sacc, sacc, cnt, wlt, bl, w1t, b1r, w2, b2r)


def kernel(x, edge_index, W_lin, b_lin, W1, b1, W2, b2):
    n, d = x.shape
    e = edge_index.shape[1]
    out_w = W2.shape[0]
    nw = NC * NS
    src3 = edge_index[0].reshape(nw, e // (nw * CH), CH)
    dst3 = edge_index[1].reshape(nw, e // (nw * CH), CH)
    zrows = jnp.zeros((n - (NS - 1) * ((n // NS) // 8 * 8), D), jnp.float32)
    sacc, cnt = _sc_scatter(x, src3, dst3, zrows)
    out_row = _tc_dense(sacc, cnt, W_lin.T, b_lin.reshape(1, D), W1.T,
                        b1.reshape(1, D), W2, b2.reshape(1, out_w))
    return out_row[0]
